# baseline (speedup 1.0000x reference)
import functools

import jax
import jax.numpy as jnp
from jax import lax
from jax.experimental import pallas as pl
from jax.experimental.pallas import tpu as pltpu
from jax.experimental.pallas import tpu_sc as plsc

N = 10000
E = 320000
D = 128
DE = 16
NMOL = 160000

NC = 2
NS = 16
NW = NC * NS
L = 16

CH = 64
NCHUNK = E // CH
CHUNKS_LO = NCHUNK // NW
CHUNKS_REM = NCHUNK - CHUNKS_LO * NW
NG = N // 8
G_LO = NG // NS
G_REM = NG - G_LO * NS

_sc_mesh = plsc.VectorSubcoreMesh(core_axis_name="c", subcore_axis_name="s")

NBUF = 3


@functools.partial(
    pl.kernel,
    out_type=jax.ShapeDtypeStruct((NC, N, D), jnp.float32),
    mesh=_sc_mesh,
    scratch_types=[
        pltpu.VMEM((CH,), jnp.int32),
        pltpu.VMEM((CH,), jnp.int32),
        pltpu.VMEM((CH,), jnp.int32),
        pltpu.VMEM((CH,), jnp.int32),
        pltpu.VMEM((CH,), jnp.int32),
        pltpu.VMEM((CH,), jnp.int32),
        pltpu.VMEM((NBUF, CH, D), jnp.float32),
        pltpu.VMEM((NBUF, CH, D), jnp.float32),
        pltpu.VMEM_SHARED((N, D), jnp.float32),
        pltpu.SemaphoreType.DMA((NBUF,)),
        pltpu.SemaphoreType.DMA((NBUF,)),
        pltpu.SemaphoreType.DMA((NBUF,)),
        pltpu.SemaphoreType.DMA((NBUF,)),
    ],
)
def _sc_layer(h_hbm, ep_hbm, src_hbm, dst_hbm, out_hbm,
              sv0, sv1, sv2, dv0, dv1, dv2, grow, epb, agg,
              gsem, esem, ssem, isem):
    c = lax.axis_index("c")
    s = lax.axis_index("s")
    wid = c * NS + s
    srcv = [sv0, sv1, sv2]
    dstv = [dv0, dv1, dv2]

    start = wid * CHUNKS_LO + jnp.minimum(wid, CHUNKS_REM)
    cnt = jnp.where(wid < CHUNKS_REM, CHUNKS_LO + 1, CHUNKS_LO)

    def issue_idx(j, b):
        pltpu.async_copy(src_hbm.at[pl.ds(j * CH, CH)], srcv[b], isem.at[b])
        pltpu.async_copy(dst_hbm.at[pl.ds(j * CH, CH)], dstv[b], isem.at[b])

    def wait_idx(b):
        pltpu.make_async_copy(src_hbm.at[pl.ds(0, CH)], srcv[b],
                              isem.at[b]).wait()
        pltpu.make_async_copy(dst_hbm.at[pl.ds(0, CH)], dstv[b],
                              isem.at[b]).wait()

    def issue_gather(b):
        pltpu.async_copy(h_hbm.at[srcv[b]], grow.at[b], gsem.at[b])

    def issue_ep(j, b):
        pltpu.async_copy(ep_hbm.at[pl.ds(j * CH, CH)], epb.at[b], esem.at[b])

    for b in range(NBUF - 1):
        issue_idx(start + b, b)
        issue_ep(start + b, b)

    zero = jnp.zeros((L,), jnp.float32)

    def zbody(r, _):
        for k in range(D // L):
            grow[NBUF - 1, r, pl.ds(k * L, L)] = zero
        return 0

    lax.fori_loop(0, CH, zbody, 0)

    base = pl.multiple_of((s * G_LO + jnp.minimum(s, G_REM)) * 8, 8)
    nlo = G_LO * 8
    rem = nlo - (nlo // CH) * CH
    for t in range(nlo // CH):
        pltpu.sync_copy(grow.at[NBUF - 1, pl.ds(0, CH)],
                        agg.at[pl.ds(base + t * CH, CH)])
    if rem:
        pltpu.sync_copy(grow.at[NBUF - 1, pl.ds(0, rem)],
                        agg.at[pl.ds(base + nlo - rem, rem)])

    @pl.when(s < G_REM)
    def _():
        pltpu.sync_copy(grow.at[NBUF - 1, pl.ds(0, 8)],
                        agg.at[pl.ds(base + nlo, 8)])

    plsc.subcore_barrier()

    wait_idx(0)
    issue_gather(0)

    n_outer = (CHUNKS_LO + 1 + NBUF - 1) // NBUF

    def outer(i, _):
        jj = i * NBUF
        for b in range(NBUF):
            j = jj + b
            bp = (b - 1) % NBUF
            bn = (b + 1) % NBUF

            @pl.when(j < cnt)
            def _():
                pltpu.make_async_copy(h_hbm.at[srcv[b]], grow.at[b],
                                      gsem.at[b]).wait()
                pltpu.make_async_copy(
                    ep_hbm.at[pl.ds(0, CH)], epb.at[b], esem.at[b]).wait()

                @plsc.parallel_loop(0, CH, step=1, unroll=4)
                def _c(r):
                    for k in range(D // L):
                        v = (grow[b, r, pl.ds(k * L, L)]
                             + epb[b, r, pl.ds(k * L, L)])
                        epb[b, r, pl.ds(k * L, L)] = jnp.maximum(v, 0.0)

                pltpu.async_copy(epb.at[b], agg.at[dstv[b]], ssem.at[b],
                                 add=True)

                @pl.when(j >= 1)
                def _w():
                    pltpu.make_async_copy(epb.at[bp], agg.at[dstv[bp]],
                                          ssem.at[bp]).wait()

                @pl.when(j + 2 < cnt)
                def _e():
                    issue_idx(start + j + 2, bp)
                    issue_ep(start + j + 2, bp)

                @pl.when(j + 1 < cnt)
                def _g():
                    wait_idx(bn)
                    issue_gather(bn)

        return 0

    lax.fori_loop(0, n_outer, outer, 0)

    for b in range(NBUF):
        @pl.when(((cnt - 1) % NBUF) == b)
        def _():
            pltpu.make_async_copy(epb.at[b], agg.at[dstv[b]],
                                  ssem.at[b]).wait()

    plsc.subcore_barrier()

    for t in range(nlo // CH):
        pltpu.sync_copy(agg.at[pl.ds(base + t * CH, CH)],
                        out_hbm.at[c, pl.ds(base + t * CH, CH)])
    if rem:
        pltpu.sync_copy(agg.at[pl.ds(base + nlo - rem, rem)],
                        out_hbm.at[c, pl.ds(base + nlo - rem, rem)])

    @pl.when(s < G_REM)
    def _():
        pltpu.sync_copy(agg.at[pl.ds(base + nlo, 8)],
                        out_hbm.at[c, pl.ds(base + nlo, 8)])


EF = E * DE // D
KW = D // DE


def _edgeproj_body(e_ref, w_ref, b_ref, o_ref):
    o_ref[...] = (
        jnp.dot(e_ref[...], w_ref[...], preferred_element_type=jnp.float32)
        + b_ref[...]
    )


def _edgeproj(e2, Wt, bt):
    BB = 1000
    return pl.pallas_call(
        _edgeproj_body,
        grid=(EF // BB,),
        in_specs=[
            pl.BlockSpec((BB, D), lambda i: (i, 0)),
            pl.BlockSpec((D, KW * D), lambda i: (0, 0)),
            pl.BlockSpec((1, KW * D), lambda i: (0, 0)),
        ],
        out_specs=pl.BlockSpec((BB, KW * D), lambda i: (i, 0)),
        out_shape=jax.ShapeDtypeStruct((EF, KW * D), jnp.float32),
    )(e2, Wt, bt.reshape(1, KW * D))


def _edge_equi_body(e_ref, f_ref, w_ref, b_ref, p_ref, o_ref):
    x = e_ref[...]
    p_ref[...] = (
        jnp.dot(x, w_ref[...], preferred_element_type=jnp.float32)
        + b_ref[...]
    )
    up = jnp.roll(x, -DE, axis=1)
    dn = jnp.roll(x, DE, axis=1)
    lane = jax.lax.broadcasted_iota(jnp.int32, x.shape, 1)
    partner = jnp.where((lane // DE) % 2 == 0, up, dn)
    o_ref[...] = x + f_ref[...] * (partner - x)


def _edge_equi(e2, F, Wt, bt):
    BB = 1000
    return pl.pallas_call(
        _edge_equi_body,
        grid=(EF // BB,),
        in_specs=[
            pl.BlockSpec((BB, D), lambda i: (i, 0)),
            pl.BlockSpec((BB, D), lambda i: (i, 0)),
            pl.BlockSpec((D, KW * D), lambda i: (0, 0)),
            pl.BlockSpec((1, KW * D), lambda i: (0, 0)),
        ],
        out_specs=[
            pl.BlockSpec((BB, KW * D), lambda i: (i, 0)),
            pl.BlockSpec((BB, D), lambda i: (i, 0)),
        ],
        out_shape=[
            jax.ShapeDtypeStruct((EF, KW * D), jnp.float32),
            jax.ShapeDtypeStruct((EF, D), jnp.float32),
        ],
    )(e2, F, Wt, bt.reshape(1, KW * D))


def _nodeup_body(h_ref, a_ref, p_ref, w_ref, bias_ref, o_ref):
    t = h_ref[...] + a_ref[...] + p_ref[...]
    t = jnp.dot(t, w_ref[...], preferred_element_type=jnp.float32) + bias_ref[...]
    o_ref[...] = jax.nn.gelu(t)


def _nodeup(h, agg_a, agg_b, W, b):
    BN = 2000
    return pl.pallas_call(
        _nodeup_body,
        grid=(N // BN,),
        in_specs=[
            pl.BlockSpec((BN, D), lambda i: (i, 0)),
            pl.BlockSpec((BN, D), lambda i: (i, 0)),
            pl.BlockSpec((BN, D), lambda i: (i, 0)),
            pl.BlockSpec((D, D), lambda i: (0, 0)),
            pl.BlockSpec((1, D), lambda i: (0, 0)),
        ],
        out_specs=pl.BlockSpec((BN, D), lambda i: (i, 0)),
        out_shape=jax.ShapeDtypeStruct((N, D), jnp.float32),
    )(h, agg_a, agg_b, W, b.reshape(1, D))


def kernel(x_h, e_h, edge_index, equi_edge_idx, nBond,
           We0, be0, W0, b0, We1, be1, W1, b1):
    src = edge_index[0]
    dst = edge_index[1]

    e2 = e_h.reshape(EF, D)
    q = equi_edge_idx.reshape(NMOL, 2)
    eqf = jnp.where(q[:, 0] == q[:, 1], jnp.float32(0.5), jnp.float32(0.0))
    F = jnp.broadcast_to(eqf[:, None], (NMOL, 2 * DE)).reshape(EF, D)

    eye = jnp.eye(KW, dtype=jnp.float32)
    Wt0 = jnp.kron(eye, We0)
    Wt1 = jnp.kron(eye, We1)
    bt0 = jnp.tile(be0, KW)
    bt1 = jnp.tile(be1, KW)

    ep0 = _edgeproj(e2, Wt0, bt0).reshape(E, D)
    agg0 = _sc_layer(x_h, ep0, src, dst)
    h1 = _nodeup(x_h, agg0[0], agg0[1], W0, b0)

    ep1f, eo = _edge_equi(e2, F, Wt1, bt1)
    ep1 = ep1f.reshape(E, D)
    agg1 = _sc_layer(h1, ep1, src, dst)
    x_out = _nodeup(h1, agg1[0], agg1[1], W1, b1)

    e_out = eo.reshape(E, DE)
    return (x_out, e_out)


# speedup vs baseline: 1.0954x; 1.0954x over previous
import functools

import jax
import jax.numpy as jnp
from jax import lax
from jax.experimental import pallas as pl
from jax.experimental.pallas import tpu as pltpu
from jax.experimental.pallas import tpu_sc as plsc

N = 10000
E = 320000
D = 128
DE = 16
NMOL = 160000

NC = 2
NS = 16
NW = NC * NS
L = 16

CH = 64
NCHUNK = E // CH
CHUNKS_LO = NCHUNK // NW
CHUNKS_REM = NCHUNK - CHUNKS_LO * NW
NG = N // 8
G_LO = NG // NS
G_REM = NG - G_LO * NS

_sc_mesh = plsc.VectorSubcoreMesh(core_axis_name="c", subcore_axis_name="s")

NBUF = 3


@functools.partial(
    pl.kernel,
    out_type=jax.ShapeDtypeStruct((NC, N, D), jnp.float32),
    mesh=_sc_mesh,
    scratch_types=[
        pltpu.VMEM((CH,), jnp.int32),
        pltpu.VMEM((CH,), jnp.int32),
        pltpu.VMEM((CH,), jnp.int32),
        pltpu.VMEM((CH,), jnp.int32),
        pltpu.VMEM((CH,), jnp.int32),
        pltpu.VMEM((CH,), jnp.int32),
        pltpu.VMEM((NBUF, CH, D), jnp.float32),
        pltpu.VMEM((NBUF, CH, D), jnp.float32),
        pltpu.VMEM_SHARED((N, D), jnp.float32),
        pltpu.SemaphoreType.DMA((NBUF,)),
        pltpu.SemaphoreType.DMA((NBUF,)),
        pltpu.SemaphoreType.DMA((NBUF,)),
        pltpu.SemaphoreType.DMA((NBUF,)),
    ],
)
def _sc_layer(h_hbm, ep_hbm, src_hbm, dst_hbm, out_hbm,
              sv0, sv1, sv2, dv0, dv1, dv2, grow, epb, agg,
              gsem, esem, ssem, isem):
    c = lax.axis_index("c")
    s = lax.axis_index("s")
    wid = c * NS + s
    srcv = [sv0, sv1, sv2]
    dstv = [dv0, dv1, dv2]

    start = wid * CHUNKS_LO + jnp.minimum(wid, CHUNKS_REM)
    cnt = jnp.where(wid < CHUNKS_REM, CHUNKS_LO + 1, CHUNKS_LO)

    def issue_idx(j, b):
        pltpu.async_copy(src_hbm.at[pl.ds(j * CH, CH)], srcv[b], isem.at[b])
        pltpu.async_copy(dst_hbm.at[pl.ds(j * CH, CH)], dstv[b], isem.at[b])

    def wait_idx(b):
        pltpu.make_async_copy(src_hbm.at[pl.ds(0, CH)], srcv[b],
                              isem.at[b]).wait()
        pltpu.make_async_copy(dst_hbm.at[pl.ds(0, CH)], dstv[b],
                              isem.at[b]).wait()

    def issue_gather(b):
        pltpu.async_copy(h_hbm.at[srcv[b]], grow.at[b], gsem.at[b])

    def issue_ep(j, b):
        pltpu.async_copy(ep_hbm.at[pl.ds(j * CH, CH)], epb.at[b], esem.at[b])

    for b in range(NBUF - 1):
        issue_idx(start + b, b)
        issue_ep(start + b, b)

    zero = jnp.zeros((L,), jnp.float32)

    def zbody(r, _):
        for k in range(D // L):
            grow[NBUF - 1, r, pl.ds(k * L, L)] = zero
        return 0

    lax.fori_loop(0, CH, zbody, 0)

    base = pl.multiple_of((s * G_LO + jnp.minimum(s, G_REM)) * 8, 8)
    nlo = G_LO * 8
    rem = nlo - (nlo // CH) * CH
    for t in range(nlo // CH):
        pltpu.sync_copy(grow.at[NBUF - 1, pl.ds(0, CH)],
                        agg.at[pl.ds(base + t * CH, CH)])
    if rem:
        pltpu.sync_copy(grow.at[NBUF - 1, pl.ds(0, rem)],
                        agg.at[pl.ds(base + nlo - rem, rem)])

    @pl.when(s < G_REM)
    def _():
        pltpu.sync_copy(grow.at[NBUF - 1, pl.ds(0, 8)],
                        agg.at[pl.ds(base + nlo, 8)])

    plsc.subcore_barrier()

    wait_idx(0)
    issue_gather(0)

    n_outer = (CHUNKS_LO + 1 + NBUF - 1) // NBUF

    def outer(i, _):
        jj = i * NBUF
        for b in range(NBUF):
            j = jj + b
            bp = (b - 1) % NBUF
            bn = (b + 1) % NBUF

            @pl.when(j < cnt)
            def _():
                pltpu.make_async_copy(h_hbm.at[srcv[b]], grow.at[b],
                                      gsem.at[b]).wait()
                pltpu.make_async_copy(
                    ep_hbm.at[pl.ds(0, CH)], epb.at[b], esem.at[b]).wait()

                @plsc.parallel_loop(0, CH, step=1, unroll=4)
                def _c(r):
                    for k in range(D // L):
                        v = (grow[b, r, pl.ds(k * L, L)]
                             + epb[b, r, pl.ds(k * L, L)])
                        epb[b, r, pl.ds(k * L, L)] = jnp.maximum(v, 0.0)

                pltpu.async_copy(epb.at[b], agg.at[dstv[b]], ssem.at[b],
                                 add=True)

                @pl.when(j >= 1)
                def _w():
                    pltpu.make_async_copy(epb.at[bp], agg.at[dstv[bp]],
                                          ssem.at[bp]).wait()

                @pl.when(j + 2 < cnt)
                def _e():
                    issue_idx(start + j + 2, bp)
                    issue_ep(start + j + 2, bp)

                @pl.when(j + 1 < cnt)
                def _g():
                    wait_idx(bn)
                    issue_gather(bn)

        return 0

    lax.fori_loop(0, n_outer, outer, 0)

    for b in range(NBUF):
        @pl.when(((cnt - 1) % NBUF) == b)
        def _():
            pltpu.make_async_copy(epb.at[b], agg.at[dstv[b]],
                                  ssem.at[b]).wait()

    plsc.subcore_barrier()

    for t in range(nlo // CH):
        pltpu.sync_copy(agg.at[pl.ds(base + t * CH, CH)],
                        out_hbm.at[c, pl.ds(base + t * CH, CH)])
    if rem:
        pltpu.sync_copy(agg.at[pl.ds(base + nlo - rem, rem)],
                        out_hbm.at[c, pl.ds(base + nlo - rem, rem)])

    @pl.when(s < G_REM)
    def _():
        pltpu.sync_copy(agg.at[pl.ds(base + nlo, 8)],
                        out_hbm.at[c, pl.ds(base + nlo, 8)])


EF = E * DE // D


def _edgeproj_body(e_ref, w_ref, b_ref, o_ref):
    o_ref[...] = (
        jnp.dot(e_ref[...], w_ref[...], preferred_element_type=jnp.float32)
        + b_ref[...]
    )


def _edgeproj(e_h, We, be):
    BE = 2000
    return pl.pallas_call(
        _edgeproj_body,
        grid=(E // BE,),
        in_specs=[
            pl.BlockSpec((BE, DE), lambda i: (i, 0)),
            pl.BlockSpec((DE, D), lambda i: (0, 0)),
            pl.BlockSpec((1, D), lambda i: (0, 0)),
        ],
        out_specs=pl.BlockSpec((BE, D), lambda i: (i, 0)),
        out_shape=jax.ShapeDtypeStruct((E, D), jnp.float32),
    )(e_h, We, be.reshape(1, D))


def _equi_body(e_ref, f_ref, o_ref):
    x = e_ref[...]
    up = jnp.roll(x, -DE, axis=1)
    dn = jnp.roll(x, DE, axis=1)
    lane = jax.lax.broadcasted_iota(jnp.int32, x.shape, 1)
    partner = jnp.where((lane // DE) % 2 == 0, up, dn)
    o_ref[...] = x + f_ref[...] * (partner - x)


def _equi(e2, F):
    BB = 2000
    return pl.pallas_call(
        _equi_body,
        grid=(EF // BB,),
        in_specs=[
            pl.BlockSpec((BB, D), lambda i: (i, 0)),
            pl.BlockSpec((BB, D), lambda i: (i, 0)),
        ],
        out_specs=pl.BlockSpec((BB, D), lambda i: (i, 0)),
        out_shape=jax.ShapeDtypeStruct((EF, D), jnp.float32),
    )(e2, F)


def _nodeup_body(h_ref, a_ref, p_ref, w_ref, bias_ref, o_ref):
    t = h_ref[...] + a_ref[...] + p_ref[...]
    t = jnp.dot(t, w_ref[...], preferred_element_type=jnp.float32) + bias_ref[...]
    o_ref[...] = jax.nn.gelu(t)


def _nodeup(h, agg_a, agg_b, W, b):
    BN = 2000
    return pl.pallas_call(
        _nodeup_body,
        grid=(N // BN,),
        in_specs=[
            pl.BlockSpec((BN, D), lambda i: (i, 0)),
            pl.BlockSpec((BN, D), lambda i: (i, 0)),
            pl.BlockSpec((BN, D), lambda i: (i, 0)),
            pl.BlockSpec((D, D), lambda i: (0, 0)),
            pl.BlockSpec((1, D), lambda i: (0, 0)),
        ],
        out_specs=pl.BlockSpec((BN, D), lambda i: (i, 0)),
        out_shape=jax.ShapeDtypeStruct((N, D), jnp.float32),
    )(h, agg_a, agg_b, W, b.reshape(1, D))


def kernel(x_h, e_h, edge_index, equi_edge_idx, nBond,
           We0, be0, W0, b0, We1, be1, W1, b1):
    src = edge_index[0]
    dst = edge_index[1]

    e2 = e_h.reshape(EF, D)
    q = equi_edge_idx.reshape(NMOL, 2)
    eqf = jnp.where(q[:, 0] == q[:, 1], jnp.float32(0.5), jnp.float32(0.0))
    F = jnp.broadcast_to(eqf[:, None], (NMOL, 2 * DE)).reshape(EF, D)

    ep0 = _edgeproj(e_h, We0, be0)
    agg0 = _sc_layer(x_h, ep0, src, dst)
    h1 = _nodeup(x_h, agg0[0], agg0[1], W0, b0)

    ep1 = _edgeproj(e_h, We1, be1)
    agg1 = _sc_layer(h1, ep1, src, dst)
    x_out = _nodeup(h1, agg1[0], agg1[1], W1, b1)

    e_out = _equi(e2, F).reshape(E, DE)
    return (x_out, e_out)


# speedup vs baseline: 1.0962x; 1.0007x over previous
import functools

import jax
import jax.numpy as jnp
from jax import lax
from jax.experimental import pallas as pl
from jax.experimental.pallas import tpu as pltpu
from jax.experimental.pallas import tpu_sc as plsc

N = 10000
E = 320000
D = 128
DE = 16
NMOL = 160000

NC = 2
NS = 16
NW = NC * NS
L = 16

CH = 64
NCHUNK = E // CH
CHUNKS_LO = NCHUNK // NW
CHUNKS_REM = NCHUNK - CHUNKS_LO * NW
NG = N // 8
G_LO = NG // NS
G_REM = NG - G_LO * NS

_sc_mesh = plsc.VectorSubcoreMesh(core_axis_name="c", subcore_axis_name="s")

NBUF = 3


@functools.partial(
    pl.kernel,
    out_type=jax.ShapeDtypeStruct((NC, N, D), jnp.float32),
    mesh=_sc_mesh,
    scratch_types=[
        pltpu.VMEM((CH,), jnp.int32),
        pltpu.VMEM((CH,), jnp.int32),
        pltpu.VMEM((CH,), jnp.int32),
        pltpu.VMEM((CH,), jnp.int32),
        pltpu.VMEM((CH,), jnp.int32),
        pltpu.VMEM((CH,), jnp.int32),
        pltpu.VMEM((NBUF, CH, D), jnp.float32),
        pltpu.VMEM((NBUF, CH, D), jnp.float32),
        pltpu.VMEM_SHARED((N, D), jnp.float32),
        pltpu.SemaphoreType.DMA((NBUF,)),
        pltpu.SemaphoreType.DMA((NBUF,)),
        pltpu.SemaphoreType.DMA((NBUF,)),
        pltpu.SemaphoreType.DMA((NBUF,)),
    ],
)
def _sc_layer(h_hbm, ep_hbm, src_hbm, dst_hbm, out_hbm,
              sv0, sv1, sv2, dv0, dv1, dv2, grow, epb, agg,
              gsem, esem, ssem, isem):
    c = lax.axis_index("c")
    s = lax.axis_index("s")
    wid = c * NS + s
    srcv = [sv0, sv1, sv2]
    dstv = [dv0, dv1, dv2]

    start = wid * CHUNKS_LO + jnp.minimum(wid, CHUNKS_REM)
    cnt = jnp.where(wid < CHUNKS_REM, CHUNKS_LO + 1, CHUNKS_LO)

    def issue_idx(j, b):
        pltpu.async_copy(src_hbm.at[pl.ds(j * CH, CH)], srcv[b], isem.at[b])
        pltpu.async_copy(dst_hbm.at[pl.ds(j * CH, CH)], dstv[b], isem.at[b])

    def wait_idx(b):
        pltpu.make_async_copy(src_hbm.at[pl.ds(0, CH)], srcv[b],
                              isem.at[b]).wait()
        pltpu.make_async_copy(dst_hbm.at[pl.ds(0, CH)], dstv[b],
                              isem.at[b]).wait()

    def issue_gather(b):
        pltpu.async_copy(h_hbm.at[srcv[b]], grow.at[b], gsem.at[b])

    def issue_ep(j, b):
        pltpu.async_copy(ep_hbm.at[pl.ds(j * CH, CH)], epb.at[b], esem.at[b])

    for b in range(NBUF - 1):
        issue_idx(start + b, b)
        issue_ep(start + b, b)

    zero = jnp.zeros((L,), jnp.float32)

    def zbody(r, _):
        for k in range(D // L):
            grow[NBUF - 1, r, pl.ds(k * L, L)] = zero
        return 0

    lax.fori_loop(0, CH, zbody, 0)

    base = pl.multiple_of((s * G_LO + jnp.minimum(s, G_REM)) * 8, 8)
    nlo = G_LO * 8
    rem = nlo - (nlo // CH) * CH
    for t in range(nlo // CH):
        pltpu.sync_copy(grow.at[NBUF - 1, pl.ds(0, CH)],
                        agg.at[pl.ds(base + t * CH, CH)])
    if rem:
        pltpu.sync_copy(grow.at[NBUF - 1, pl.ds(0, rem)],
                        agg.at[pl.ds(base + nlo - rem, rem)])

    @pl.when(s < G_REM)
    def _():
        pltpu.sync_copy(grow.at[NBUF - 1, pl.ds(0, 8)],
                        agg.at[pl.ds(base + nlo, 8)])

    plsc.subcore_barrier()

    wait_idx(0)
    issue_gather(0)

    n_outer = (CHUNKS_LO + 1 + NBUF - 1) // NBUF

    def outer(i, _):
        jj = i * NBUF
        for b in range(NBUF):
            j = jj + b
            bp = (b - 1) % NBUF
            bn = (b + 1) % NBUF

            @pl.when(j < cnt)
            def _():
                pltpu.make_async_copy(h_hbm.at[srcv[b]], grow.at[b],
                                      gsem.at[b]).wait()
                pltpu.make_async_copy(
                    ep_hbm.at[pl.ds(0, CH)], epb.at[b], esem.at[b]).wait()

                @plsc.parallel_loop(0, CH, step=1, unroll=4)
                def _c(r):
                    for k in range(D // L):
                        v = (grow[b, r, pl.ds(k * L, L)]
                             + epb[b, r, pl.ds(k * L, L)])
                        epb[b, r, pl.ds(k * L, L)] = jnp.maximum(v, 0.0)

                pltpu.async_copy(epb.at[b], agg.at[dstv[b]], ssem.at[b],
                                 add=True)

                @pl.when(j >= 1)
                def _w():
                    pltpu.make_async_copy(epb.at[bp], agg.at[dstv[bp]],
                                          ssem.at[bp]).wait()

                @pl.when(j + 2 < cnt)
                def _e():
                    issue_idx(start + j + 2, bp)
                    issue_ep(start + j + 2, bp)

                @pl.when(j + 1 < cnt)
                def _g():
                    wait_idx(bn)
                    issue_gather(bn)

        return 0

    lax.fori_loop(0, n_outer, outer, 0)

    for b in range(NBUF):
        @pl.when(((cnt - 1) % NBUF) == b)
        def _():
            pltpu.make_async_copy(epb.at[b], agg.at[dstv[b]],
                                  ssem.at[b]).wait()

    plsc.subcore_barrier()

    for t in range(nlo // CH):
        pltpu.sync_copy(agg.at[pl.ds(base + t * CH, CH)],
                        out_hbm.at[c, pl.ds(base + t * CH, CH)])
    if rem:
        pltpu.sync_copy(agg.at[pl.ds(base + nlo - rem, rem)],
                        out_hbm.at[c, pl.ds(base + nlo - rem, rem)])

    @pl.when(s < G_REM)
    def _():
        pltpu.sync_copy(agg.at[pl.ds(base + nlo, 8)],
                        out_hbm.at[c, pl.ds(base + nlo, 8)])


EF = E * DE // D


def _edgeproj_body(e_ref, w_ref, b_ref, o_ref):
    o_ref[...] = (
        jnp.dot(e_ref[...], w_ref[...], preferred_element_type=jnp.float32)
        + b_ref[...]
    )


def _edgeproj(e_h, We, be):
    BE = 2000
    return pl.pallas_call(
        _edgeproj_body,
        grid=(E // BE,),
        in_specs=[
            pl.BlockSpec((BE, DE), lambda i: (i, 0)),
            pl.BlockSpec((DE, D), lambda i: (0, 0)),
            pl.BlockSpec((1, D), lambda i: (0, 0)),
        ],
        out_specs=pl.BlockSpec((BE, D), lambda i: (i, 0)),
        out_shape=jax.ShapeDtypeStruct((E, D), jnp.float32),
    )(e_h, We, be.reshape(1, D))


def _equi_body(e_ref, f_ref, o_ref):
    x = e_ref[...]
    up = jnp.roll(x, -DE, axis=1)
    dn = jnp.roll(x, DE, axis=1)
    lane = jax.lax.broadcasted_iota(jnp.int32, x.shape, 1)
    partner = jnp.where((lane // DE) % 2 == 0, up, dn)
    o_ref[...] = x + f_ref[...] * (partner - x)


def _equi(e2, F):
    BB = 2000
    return pl.pallas_call(
        _equi_body,
        grid=(EF // BB,),
        in_specs=[
            pl.BlockSpec((BB, D), lambda i: (i, 0)),
            pl.BlockSpec((BB, D), lambda i: (i, 0)),
        ],
        out_specs=pl.BlockSpec((BB, D), lambda i: (i, 0)),
        out_shape=jax.ShapeDtypeStruct((EF, D), jnp.float32),
    )(e2, F)


def _nodeup_body(h_ref, a_ref, p_ref, w_ref, bias_ref, o_ref):
    t = h_ref[...] + a_ref[...] + p_ref[...]
    t = jnp.dot(t, w_ref[...], preferred_element_type=jnp.float32) + bias_ref[...]
    o_ref[...] = jax.nn.gelu(t)


def _nodeup(h, agg_a, agg_b, W, b):
    BN = 2000
    return pl.pallas_call(
        _nodeup_body,
        grid=(N // BN,),
        in_specs=[
            pl.BlockSpec((BN, D), lambda i: (i, 0)),
            pl.BlockSpec((BN, D), lambda i: (i, 0)),
            pl.BlockSpec((BN, D), lambda i: (i, 0)),
            pl.BlockSpec((D, D), lambda i: (0, 0)),
            pl.BlockSpec((1, D), lambda i: (0, 0)),
        ],
        out_specs=pl.BlockSpec((BN, D), lambda i: (i, 0)),
        out_shape=jax.ShapeDtypeStruct((N, D), jnp.float32),
    )(h, agg_a, agg_b, W, b.reshape(1, D))


def kernel(x_h, e_h, edge_index, equi_edge_idx, nBond,
           We0, be0, W0, b0, We1, be1, W1, b1):
    src = edge_index[0]
    dst = edge_index[1]

    e2 = e_h.reshape(EF, D)
    q = equi_edge_idx.reshape(NMOL, 2)
    eqf = jnp.where(q[:, 0] == q[:, 1], jnp.float32(0.5), jnp.float32(0.0))
    F = jnp.broadcast_to(eqf[:, None], (NMOL, 2 * DE)).reshape(EF, D)

    ep0 = _edgeproj(e_h, We0, be0)
    agg0 = _sc_layer(x_h, ep0, src, dst)
    ep1 = _edgeproj(e_h, We1, be1)
    e_out = _equi(e2, F).reshape(E, DE)

    h1 = _nodeup(x_h, agg0[0], agg0[1], W0, b0)
    agg1 = _sc_layer(h1, ep1, src, dst)
    x_out = _nodeup(h1, agg1[0], agg1[1], W1, b1)
    return (x_out, e_out)


# speedup vs baseline: 1.2347x; 1.1263x over previous
import functools

import jax
import jax.numpy as jnp
from jax import lax
from jax.experimental import pallas as pl
from jax.experimental.pallas import tpu as pltpu
from jax.experimental.pallas import tpu_sc as plsc

N = 10000
E = 320000
D = 128
DE = 16
NMOL = 160000

NC = 2
NS = 16
NW = NC * NS
L = 16

CH = 64
NCHUNK = E // CH
CHUNKS_LO = NCHUNK // NW
CHUNKS_REM = NCHUNK - CHUNKS_LO * NW
NG = N // 8
G_LO = NG // NS
G_REM = NG - G_LO * NS

_sc_mesh = plsc.VectorSubcoreMesh(core_axis_name="c", subcore_axis_name="s")

NBUF = 3


@functools.partial(
    pl.kernel,
    out_type=jax.ShapeDtypeStruct((NC, N, D), jnp.float32),
    mesh=_sc_mesh,
    scratch_types=[
        pltpu.VMEM((CH,), jnp.int32),
        pltpu.VMEM((CH,), jnp.int32),
        pltpu.VMEM((CH,), jnp.int32),
        pltpu.VMEM((CH,), jnp.int32),
        pltpu.VMEM((CH,), jnp.int32),
        pltpu.VMEM((CH,), jnp.int32),
        pltpu.VMEM((NBUF, CH, D), jnp.float32),
        pltpu.VMEM((NBUF, CH, D), jnp.float32),
        pltpu.VMEM_SHARED((N, D), jnp.float32),
        pltpu.SemaphoreType.DMA((NBUF,)),
        pltpu.SemaphoreType.DMA((NBUF,)),
        pltpu.SemaphoreType.DMA((NBUF,)),
        pltpu.SemaphoreType.DMA((NBUF,)),
    ],
)
def _sc_layer(h_hbm, ep_hbm, src_hbm, dst_hbm, out_hbm,
              sv0, sv1, sv2, dv0, dv1, dv2, grow, epb, agg,
              gsem, esem, ssem, isem):
    c = lax.axis_index("c")
    s = lax.axis_index("s")
    wid = c * NS + s
    srcv = [sv0, sv1, sv2]
    dstv = [dv0, dv1, dv2]

    start = wid * CHUNKS_LO + jnp.minimum(wid, CHUNKS_REM)
    cnt = jnp.where(wid < CHUNKS_REM, CHUNKS_LO + 1, CHUNKS_LO)

    def issue_idx(j, b):
        pltpu.async_copy(src_hbm.at[pl.ds(j * CH, CH)], srcv[b], isem.at[b])
        pltpu.async_copy(dst_hbm.at[pl.ds(j * CH, CH)], dstv[b], isem.at[b])

    def wait_idx(b):
        pltpu.make_async_copy(src_hbm.at[pl.ds(0, CH)], srcv[b],
                              isem.at[b]).wait()
        pltpu.make_async_copy(dst_hbm.at[pl.ds(0, CH)], dstv[b],
                              isem.at[b]).wait()

    def issue_gather(b):
        pltpu.async_copy(h_hbm.at[srcv[b]], grow.at[b], gsem.at[b])

    def issue_ep(j, b):
        pltpu.async_copy(ep_hbm.at[pl.ds(j * CH, CH)], epb.at[b], esem.at[b])

    for b in range(NBUF - 1):
        issue_idx(start + b, b)
        issue_ep(start + b, b)

    zero = jnp.zeros((L,), jnp.float32)

    def zbody(r, _):
        for k in range(D // L):
            grow[NBUF - 1, r, pl.ds(k * L, L)] = zero
        return 0

    lax.fori_loop(0, CH, zbody, 0)

    base = pl.multiple_of((s * G_LO + jnp.minimum(s, G_REM)) * 8, 8)
    nlo = G_LO * 8
    rem = nlo - (nlo // CH) * CH
    for t in range(nlo // CH):
        pltpu.sync_copy(grow.at[NBUF - 1, pl.ds(0, CH)],
                        agg.at[pl.ds(base + t * CH, CH)])
    if rem:
        pltpu.sync_copy(grow.at[NBUF - 1, pl.ds(0, rem)],
                        agg.at[pl.ds(base + nlo - rem, rem)])

    @pl.when(s < G_REM)
    def _():
        pltpu.sync_copy(grow.at[NBUF - 1, pl.ds(0, 8)],
                        agg.at[pl.ds(base + nlo, 8)])

    plsc.subcore_barrier()

    wait_idx(0)
    issue_gather(0)

    n_outer = (CHUNKS_LO + 1 + NBUF - 1) // NBUF

    def outer(i, _):
        jj = i * NBUF
        for b in range(NBUF):
            j = jj + b
            bp = (b - 1) % NBUF
            bn = (b + 1) % NBUF

            @pl.when(j < cnt)
            def _():
                pltpu.make_async_copy(h_hbm.at[srcv[b]], grow.at[b],
                                      gsem.at[b]).wait()
                pltpu.make_async_copy(
                    ep_hbm.at[pl.ds(0, CH)], epb.at[b], esem.at[b]).wait()

                @plsc.parallel_loop(0, CH, step=1, unroll=4)
                def _c(r):
                    for k in range(D // L):
                        v = (grow[b, r, pl.ds(k * L, L)]
                             + epb[b, r, pl.ds(k * L, L)])
                        epb[b, r, pl.ds(k * L, L)] = jnp.maximum(v, 0.0)

                pltpu.async_copy(epb.at[b], agg.at[dstv[b]], ssem.at[b],
                                 add=True)

                @pl.when(j >= 1)
                def _w():
                    pltpu.make_async_copy(epb.at[bp], agg.at[dstv[bp]],
                                          ssem.at[bp]).wait()

                @pl.when(j + 2 < cnt)
                def _e():
                    issue_idx(start + j + 2, bp)
                    issue_ep(start + j + 2, bp)

                @pl.when(j + 1 < cnt)
                def _g():
                    wait_idx(bn)
                    issue_gather(bn)

        return 0

    lax.fori_loop(0, n_outer, outer, 0)

    for b in range(NBUF):
        @pl.when(((cnt - 1) % NBUF) == b)
        def _():
            pltpu.make_async_copy(epb.at[b], agg.at[dstv[b]],
                                  ssem.at[b]).wait()

    plsc.subcore_barrier()

    for t in range(nlo // CH):
        pltpu.sync_copy(agg.at[pl.ds(base + t * CH, CH)],
                        out_hbm.at[c, pl.ds(base + t * CH, CH)])
    if rem:
        pltpu.sync_copy(agg.at[pl.ds(base + nlo - rem, rem)],
                        out_hbm.at[c, pl.ds(base + nlo - rem, rem)])

    @pl.when(s < G_REM)
    def _():
        pltpu.sync_copy(agg.at[pl.ds(base + nlo, 8)],
                        out_hbm.at[c, pl.ds(base + nlo, 8)])


EF = E * DE // D


KW = D // DE


def _edgeproj_body(e_ref, w_ref, b_ref, o_ref):
    res = (
        jnp.dot(e_ref[...], w_ref[...], preferred_element_type=jnp.float32)
        + b_ref[...]
    )
    for k in range(KW):
        o_ref[:, k, :] = res[:, k * D:(k + 1) * D]


def _edgeproj(e2, Wt, bt):
    BB = 1000
    return pl.pallas_call(
        _edgeproj_body,
        grid=(EF // BB,),
        in_specs=[
            pl.BlockSpec((BB, D), lambda i: (i, 0)),
            pl.BlockSpec((D, KW * D), lambda i: (0, 0)),
            pl.BlockSpec((1, KW * D), lambda i: (0, 0)),
        ],
        out_specs=pl.BlockSpec((BB, KW, D), lambda i: (i, 0, 0)),
        out_shape=jax.ShapeDtypeStruct((EF, KW, D), jnp.float32),
    )(e2, Wt, bt.reshape(1, KW * D))


def _equi_body(e_ref, f_ref, o_ref):
    x = e_ref[...]
    up = jnp.roll(x, -DE, axis=1)
    dn = jnp.roll(x, DE, axis=1)
    lane = jax.lax.broadcasted_iota(jnp.int32, x.shape, 1)
    partner = jnp.where((lane // DE) % 2 == 0, up, dn)
    o_ref[...] = x + f_ref[...] * (partner - x)


def _equi(e2, F):
    BB = 2000
    return pl.pallas_call(
        _equi_body,
        grid=(EF // BB,),
        in_specs=[
            pl.BlockSpec((BB, D), lambda i: (i, 0)),
            pl.BlockSpec((BB, D), lambda i: (i, 0)),
        ],
        out_specs=pl.BlockSpec((BB, D), lambda i: (i, 0)),
        out_shape=jax.ShapeDtypeStruct((EF, D), jnp.float32),
    )(e2, F)


def _nodeup_body(h_ref, a_ref, p_ref, w_ref, bias_ref, o_ref):
    t = h_ref[...] + a_ref[...] + p_ref[...]
    t = jnp.dot(t, w_ref[...], preferred_element_type=jnp.float32) + bias_ref[...]
    o_ref[...] = jax.nn.gelu(t)


def _nodeup(h, agg_a, agg_b, W, b):
    BN = 2000
    return pl.pallas_call(
        _nodeup_body,
        grid=(N // BN,),
        in_specs=[
            pl.BlockSpec((BN, D), lambda i: (i, 0)),
            pl.BlockSpec((BN, D), lambda i: (i, 0)),
            pl.BlockSpec((BN, D), lambda i: (i, 0)),
            pl.BlockSpec((D, D), lambda i: (0, 0)),
            pl.BlockSpec((1, D), lambda i: (0, 0)),
        ],
        out_specs=pl.BlockSpec((BN, D), lambda i: (i, 0)),
        out_shape=jax.ShapeDtypeStruct((N, D), jnp.float32),
    )(h, agg_a, agg_b, W, b.reshape(1, D))


def kernel(x_h, e_h, edge_index, equi_edge_idx, nBond,
           We0, be0, W0, b0, We1, be1, W1, b1):
    src = edge_index[0]
    dst = edge_index[1]

    e2 = e_h.reshape(EF, D)
    q = equi_edge_idx.reshape(NMOL, 2)
    eqf = jnp.where(q[:, 0] == q[:, 1], jnp.float32(0.5), jnp.float32(0.0))
    F = jnp.broadcast_to(eqf[:, None], (NMOL, 2 * DE)).reshape(EF, D)

    eye = jnp.eye(KW, dtype=jnp.float32)
    Wt0 = jnp.kron(eye, We0)
    Wt1 = jnp.kron(eye, We1)
    bt0 = jnp.tile(be0, KW)
    bt1 = jnp.tile(be1, KW)

    ep0 = _edgeproj(e2, Wt0, bt0).reshape(E, D)
    agg0 = _sc_layer(x_h, ep0, src, dst)
    ep1 = _edgeproj(e2, Wt1, bt1).reshape(E, D)
    e_out = _equi(e2, F).reshape(E, DE)

    h1 = _nodeup(x_h, agg0[0], agg0[1], W0, b0)
    agg1 = _sc_layer(h1, ep1, src, dst)
    x_out = _nodeup(h1, agg1[0], agg1[1], W1, b1)
    return (x_out, e_out)
